# baseline (speedup 1.0000x reference)
import functools

import jax
import jax.numpy as jnp
from jax import lax
from jax.experimental import pallas as pl
from jax.experimental.pallas import tpu as pltpu
from jax.experimental.pallas import tpu_sc as plsc

NC = 2
NS = 16
L = 16
NW = NC * NS

B = 16384
D = 64
CHUNK = 128
BPW = B // NW
KCH = BPW // CHUNK
JG = D // L

_mesh = plsc.VectorSubcoreMesh(core_axis_name="c", subcore_axis_name="s")


@functools.partial(
    pl.kernel,
    mesh=_mesh,
    out_type=jax.ShapeDtypeStruct((NC, L), jnp.float32),
    compiler_params=pltpu.CompilerParams(use_tc_tiling_on_sc=False),
    scratch_types=[
        pltpu.VMEM((KCH, CHUNK), jnp.int32),
        pltpu.VMEM((KCH, CHUNK, D), jnp.float32),
        pltpu.VMEM((KCH, CHUNK, D), jnp.float32),
        pltpu.VMEM((L,), jnp.float32),
        pltpu.VMEM_SHARED((NS, L), jnp.float32),
        pltpu.VMEM((NS, L), jnp.float32),
        pltpu.SemaphoreType.DMA,
        pltpu.SemaphoreType.DMA,
    ],
)
def _center_loss_sc(z_hbm, lab_hbm, tab_hbm, out_hbm,
                    idx_v, rows_v, z_v, acc_v, shared, red_v, sem, zsem):
    c = lax.axis_index("c")
    s = lax.axis_index("s")
    wid = c * NS + s

    pltpu.sync_copy(lab_hbm.at[wid], idx_v)
    zcp = pltpu.async_copy(z_hbm.at[wid], z_v, zsem)
    gathers = [
        pltpu.async_copy(tab_hbm.at[idx_v.at[k]], rows_v.at[k], sem)
        for k in range(KCH)
    ]
    zcp.wait()
    for cp in gathers:
        cp.wait()

    accs = (jnp.zeros((L,), jnp.float32),) * JG
    for k in range(KCH):
        def row_step(r, a, k=k):
            out = []
            for j in range(JG):
                zv = z_v[k, r, pl.ds(j * L, L)]
                cv = rows_v[k, r, pl.ds(j * L, L)]
                d = zv - cv
                out.append(a[j] + d * d)
            return tuple(out)
        accs = lax.fori_loop(0, CHUNK, row_step, accs)

    tot = accs[0]
    for j in range(1, JG):
        tot = tot + accs[j]
    acc_v[...] = tot * (1.0 / B)

    pltpu.sync_copy(acc_v, shared.at[s])
    plsc.subcore_barrier()

    @pl.when(s == 0)
    def _():
        pltpu.sync_copy(shared, red_v)
        core_tot = red_v[0]
        for i in range(1, NS):
            core_tot = core_tot + red_v[i]
        acc_v[...] = core_tot
        pltpu.sync_copy(acc_v, out_hbm.at[c])


def kernel(z, labels, centers):
    lab = labels.astype(jnp.int32).reshape(NW, KCH, CHUNK)
    zr = z.reshape(NW, KCH, CHUNK, D)
    partials = _center_loss_sc(zr, lab, centers)
    return jnp.sum(partials)


# speedup vs baseline: 1.1173x; 1.1173x over previous
import functools

import jax
import jax.numpy as jnp
from jax import lax
from jax.experimental import pallas as pl
from jax.experimental.pallas import tpu as pltpu
from jax.experimental.pallas import tpu_sc as plsc

NC = 2
NS = 16
L = 16
NW = NC * NS

B = 16384
D = 64
W = 128
CHUNK = 128
BPW = B // NW
KCH = BPW // CHUNK
JG = D // L

_mesh = plsc.VectorSubcoreMesh(core_axis_name="c", subcore_axis_name="s")


@functools.partial(
    pl.kernel,
    mesh=_mesh,
    out_type=jax.ShapeDtypeStruct((NC, L), jnp.float32),
    compiler_params=pltpu.CompilerParams(use_tc_tiling_on_sc=True),
    scratch_types=[
        pltpu.VMEM((KCH, CHUNK), jnp.int32),
        pltpu.VMEM((KCH, CHUNK, W), jnp.float32),
        pltpu.VMEM((KCH, CHUNK // 2, W), jnp.float32),
        pltpu.VMEM((L,), jnp.float32),
        pltpu.VMEM_SHARED((NS, L), jnp.float32),
        pltpu.VMEM((NS, L), jnp.float32),
        pltpu.SemaphoreType.DMA,
        pltpu.SemaphoreType.DMA,
    ],
)
def _center_loss_sc(z_hbm, lab_hbm, tab_hbm, out_hbm,
                    idx_v, rows_v, z_v, acc_v, shared, red_v, sem, zsem):
    c = lax.axis_index("c")
    s = lax.axis_index("s")
    wid = c * NS + s

    pltpu.sync_copy(lab_hbm.at[wid], idx_v)
    zcp = pltpu.async_copy(z_hbm.at[wid], z_v, zsem)
    gathers = [
        pltpu.async_copy(tab_hbm.at[idx_v.at[k]], rows_v.at[k], sem)
        for k in range(KCH)
    ]
    zcp.wait()
    for cp in gathers:
        cp.wait()

    accs = (jnp.zeros((L,), jnp.float32),) * JG
    for k in range(KCH):
        def row_step(i, a, k=k):
            out = list(a)
            for p in range(2):
                for j in range(JG):
                    zv = z_v[k, i, pl.ds(p * D + j * L, L)]
                    cv = rows_v[k, 2 * i + p, pl.ds(j * L, L)]
                    d = zv - cv
                    out[j] = out[j] + d * d
            return tuple(out)
        accs = lax.fori_loop(0, CHUNK // 2, row_step, accs)

    tot = accs[0]
    for j in range(1, JG):
        tot = tot + accs[j]
    acc_v[...] = tot * (1.0 / B)

    pltpu.sync_copy(acc_v, shared.at[s])
    plsc.subcore_barrier()

    @pl.when(s == 0)
    def _():
        pltpu.sync_copy(shared, red_v)
        core_tot = red_v[0]
        for i in range(1, NS):
            core_tot = core_tot + red_v[i]
        acc_v[...] = core_tot
        pltpu.sync_copy(acc_v, out_hbm.at[c])


def kernel(z, labels, centers):
    n = centers.shape[0]
    tab = jnp.pad(centers, ((0, 0), (0, W - D)))
    lab = labels.astype(jnp.int32).reshape(NW, KCH, CHUNK)
    zr = z.reshape(NW, KCH, CHUNK // 2, W)
    partials = _center_loss_sc(zr, lab, tab)
    return jnp.sum(partials)
